# baseline (speedup 1.0000x reference)
import jax
import jax.numpy as jnp
from jax.experimental import pallas as pl
from jax.experimental.pallas import tpu as pltpu

_NBUF = 3


def _pick_block_rows(n: int) -> int:
    best = 8
    for bm in range(8, min(n, 40) + 1, 8):
        if n % bm == 0:
            best = bm
    return best


def _copy_in(adj_ref, abuf, sem, idx, slot, bm):
    return pltpu.make_async_copy(
        adj_ref.at[pl.ds(idx * bm, bm), :],
        abuf.at[slot],
        sem.at[slot],
    )


def _gc_kernel(x_ref, w_ref, b_ref, adj_ref, out_ref, abuf, h_ref, sem):
    i = pl.program_id(0)
    t = pl.num_programs(0)
    bm = abuf.shape[1]

    lookahead = _NBUF - 1

    @pl.when(i == 0)
    def _prologue():
        for j in range(lookahead):
            _copy_in(adj_ref, abuf, sem, j, j, bm).start()
        h_ref[...] = jax.lax.dot_general(
            x_ref[...], w_ref[...],
            (((1,), (1,)), ((), ())),
            preferred_element_type=jnp.float32,
        ) + b_ref[...]

    @pl.when(i + lookahead < t)
    def _prefetch():
        _copy_in(adj_ref, abuf, sem, i + lookahead,
                 (i + lookahead) % _NBUF, bm).start()

    slot = i % _NBUF
    _copy_in(adj_ref, abuf, sem, i, slot, bm).wait()
    out_ref[...] = jnp.dot(
        abuf[slot], h_ref[...],
        preferred_element_type=jnp.float32,
    )


def kernel(x, adj, W, b):
    n, d_in = x.shape
    d_out = W.shape[0]
    bm = _pick_block_rows(n)
    grid = (n // bm,)
    return pl.pallas_call(
        _gc_kernel,
        grid=grid,
        in_specs=[
            pl.BlockSpec((n, d_in), lambda i: (0, 0)),
            pl.BlockSpec((d_out, d_in), lambda i: (0, 0)),
            pl.BlockSpec((1, d_out), lambda i: (0, 0)),
            pl.BlockSpec(memory_space=pl.ANY),
        ],
        out_specs=pl.BlockSpec((bm, d_out), lambda i: (i, 0)),
        out_shape=jax.ShapeDtypeStruct((n, d_out), jnp.float32),
        scratch_shapes=[
            pltpu.VMEM((_NBUF, bm, n), jnp.float32),
            pltpu.VMEM((n, d_out), jnp.float32),
            pltpu.SemaphoreType.DMA((_NBUF,)),
        ],
        compiler_params=pltpu.CompilerParams(
            dimension_semantics=("arbitrary",),
            vmem_limit_bytes=100 * 1024 * 1024,
        ),
    )(x, W, b.reshape(1, -1), adj)


# speedup vs baseline: 1.4900x; 1.4900x over previous
import jax
import jax.numpy as jnp
from jax.experimental import pallas as pl
from jax.experimental.pallas import tpu as pltpu

_NBUF = 4


def _pick_block_rows(n: int) -> int:
    best = 8
    for bm in range(8, min(n, 80) + 1, 8):
        if n % bm == 0:
            best = bm
    return best


def _copy_in(adj_ref, abuf, sem, idx, slot, bm):
    return pltpu.make_async_copy(
        adj_ref.at[pl.ds(idx * bm, bm), :],
        abuf.at[slot],
        sem.at[slot],
    )


def _gc_kernel(x_ref, w_ref, b_ref, adj_ref, out_ref, abuf, h_ref, sem):
    i = pl.program_id(0)
    t = pl.num_programs(0)
    bm = abuf.shape[1]

    lookahead = _NBUF - 1

    @pl.when(i == 0)
    def _prologue():
        for j in range(lookahead):
            _copy_in(adj_ref, abuf, sem, j, j, bm).start()
        h_ref[...] = jax.lax.dot_general(
            x_ref[...], w_ref[...],
            (((1,), (1,)), ((), ())),
            preferred_element_type=jnp.float32,
        ) + b_ref[...]

    @pl.when(i + lookahead < t)
    def _prefetch():
        _copy_in(adj_ref, abuf, sem, i + lookahead,
                 (i + lookahead) % _NBUF, bm).start()

    slot = i % _NBUF
    _copy_in(adj_ref, abuf, sem, i, slot, bm).wait()
    out_ref[...] = jnp.dot(
        abuf[slot], h_ref[...],
        preferred_element_type=jnp.float32,
    )


def kernel(x, adj, W, b):
    n, d_in = x.shape
    d_out = W.shape[0]
    bm = _pick_block_rows(n)
    grid = (n // bm,)
    return pl.pallas_call(
        _gc_kernel,
        grid=grid,
        in_specs=[
            pl.BlockSpec((n, d_in), lambda i: (0, 0)),
            pl.BlockSpec((d_out, d_in), lambda i: (0, 0)),
            pl.BlockSpec((1, d_out), lambda i: (0, 0)),
            pl.BlockSpec(memory_space=pl.ANY),
        ],
        out_specs=pl.BlockSpec((bm, d_out), lambda i: (i, 0)),
        out_shape=jax.ShapeDtypeStruct((n, d_out), jnp.float32),
        scratch_shapes=[
            pltpu.VMEM((_NBUF, bm, n), jnp.float32),
            pltpu.VMEM((n, d_out), jnp.float32),
            pltpu.SemaphoreType.DMA((_NBUF,)),
        ],
        compiler_params=pltpu.CompilerParams(
            dimension_semantics=("arbitrary",),
            vmem_limit_bytes=100 * 1024 * 1024,
        ),
    )(x, W, b.reshape(1, -1), adj)
